# baseline (speedup 1.0000x reference)
import jax
import jax.numpy as jnp
from jax.experimental import pallas as pl


_N = 3000000
_BS = 262144
_G = -(-_N // _BS)


def _sumsq_body(inp_ref, tgt_ref, out_ref):
    i = pl.program_id(0)

    @pl.when(i == 0)
    def _init():
        out_ref[...] = jnp.zeros_like(out_ref)

    d = inp_ref[...] - tgt_ref[...]
    sq = d * d

    @pl.when(i < _G - 1)
    def _full():
        out_ref[...] += jnp.sum(sq).reshape(1, 1)

    @pl.when(i == _G - 1)
    def _ragged():
        idx = jax.lax.broadcasted_iota(jnp.int32, (_BS,), 0)
        valid = idx < (_N - (_G - 1) * _BS)
        out_ref[...] += jnp.sum(jnp.where(valid, sq, 0.0)).reshape(1, 1)


def kernel(input, target, mult_mask, natoms, step):
    del mult_mask, natoms, step
    inp = input.reshape(_N)
    tgt = target.reshape(_N)
    out = pl.pallas_call(
        _sumsq_body,
        grid=(_G,),
        in_specs=[
            pl.BlockSpec((_BS,), lambda i: (i,)),
            pl.BlockSpec((_BS,), lambda i: (i,)),
        ],
        out_specs=pl.BlockSpec((1, 1), lambda i: (0, 0)),
        out_shape=jax.ShapeDtypeStruct((1, 1), jnp.float32),
    )(inp, tgt)
    return out[0, 0]
